# Initial kernel scaffold; baseline (speedup 1.0000x reference)
import jax
import jax.numpy as jnp
from jax.experimental import pallas as pl


def kernel(si, sj, t, e, edge_src, edge_dst, basis_freq, phase, w_ih, w_hh, b_ih, b_hh):
    raise NotImplementedError("write your pallas kernel here")


# speedup vs baseline: 8.1816x; 8.1816x over previous
import dataclasses
import functools

import jax
import jax.numpy as jnp
from jax import lax
from jax.experimental import pallas as pl
from jax.experimental.pallas import tpu as pltpu
from jax.experimental.pallas import tpu_sc as plsc

N_NODES = 5000
E_TOTAL = 160000
D_S = 128
D_T = 16
D_E = 16

NW = 32
EPW = E_TOTAL // NW
NPAD = 5120
NPW = NPAD // NW

_vmesh = plsc.VectorSubcoreMesh(core_axis_name="c", subcore_axis_name="s")

_sc_params = pltpu.CompilerParams()
if "needs_layout_passes" in pltpu.CompilerParams.__dataclass_fields__:
    _sc_params = dataclasses.replace(_sc_params, needs_layout_passes=False)
if "use_tc_tiling_on_sc" in pltpu.CompilerParams.__dataclass_fields__:
    _sc_params = dataclasses.replace(_sc_params, use_tc_tiling_on_sc=False)


@jax.jit
def _sc_scatter(edge_src, edge_dst):
    @functools.partial(
        pl.kernel,
        out_type=[
            jax.ShapeDtypeStruct((NW, NPAD), jnp.int32),
            jax.ShapeDtypeStruct((NW, NPAD), jnp.int32),
        ],
        mesh=_vmesh,
        compiler_params=_sc_params,
        scratch_types=[
            pltpu.VMEM((EPW + 16,), jnp.int32),
            pltpu.VMEM((EPW + 16,), jnp.int32),
            pltpu.VMEM((NPAD,), jnp.int32),
            pltpu.VMEM((NPAD,), jnp.int32),
            pltpu.VMEM((32,), jnp.int32),
        ],
    )
    def k(src_hbm, dst_hbm, pr_hbm, pg_hbm, src_v, dst_v, lr_v, lg_v,
          shift_v):
        wid = lax.axis_index("s") * 2 + lax.axis_index("c")
        base = wid * EPW
        pltpu.sync_copy(src_hbm.at[pl.ds(base, EPW)],
                        src_v.at[pl.ds(0, EPW)])
        pltpu.sync_copy(dst_hbm.at[pl.ds(base, EPW)],
                        dst_v.at[pl.ds(0, EPW)])

        neg = jnp.full((16,), -1, jnp.int32)

        @pl.loop(0, NPAD, step=16)
        def _(i):
            lr_v[pl.ds(i, 16)] = neg
            lg_v[pl.ds(i, 16)] = neg

        iota16 = lax.iota(jnp.int32, 16)
        shift_v[pl.ds(16, 16)] = neg

        def scatter_one(arr_ref, idxv, val):
            comp = (idxv << 18) | val
            sk, sval = plsc.sort_key_val(comp, val)
            shift_v[pl.ds(0, 16)] = sk
            nxt = shift_v[pl.ds(1, 16)]
            is_last = (sk >> 18) != (nxt >> 18)
            plsc.store_scatter(arr_ref, [sk >> 18], sval, mask=is_last)

        def do_vreg(k_, valid):
            sv = src_v[pl.ds(k_, 16)]
            dv = dst_v[pl.ds(k_, 16)]
            val = base + k_ + iota16
            if valid is not None:
                sv = jnp.where(valid, sv, NPAD - 1)
                dv = jnp.where(valid, dv, NPAD - 1)
                val = jnp.where(valid, val, 0)
            scatter_one(lr_v, sv, val)
            scatter_one(lg_v, dv, val)

        tail_n = EPW % 16
        main_hi = EPW - tail_n

        @pl.loop(0, main_hi, step=16)
        def _(k_):
            do_vreg(k_, None)

        if tail_n:
            do_vreg(main_hi, iota16 < tail_n)

        pltpu.sync_copy(lr_v, pr_hbm.at[wid])
        pltpu.sync_copy(lg_v, pg_hbm.at[wid])

    return k(edge_src, edge_dst)


@jax.jit
def _sc_gather(part_r, part_g, edge_src, edge_dst, t, e, si, sj):
    out_type = [
        jax.ShapeDtypeStruct((NPAD,), jnp.float32),
        jax.ShapeDtypeStruct((NPAD,), jnp.float32),
        jax.ShapeDtypeStruct((NPAD, D_E), jnp.float32),
        jax.ShapeDtypeStruct((NPAD, D_S), jnp.float32),
        jax.ShapeDtypeStruct((NPAD,), jnp.float32),
        jax.ShapeDtypeStruct((NPAD,), jnp.float32),
        jax.ShapeDtypeStruct((NPAD, D_E), jnp.float32),
        jax.ShapeDtypeStruct((NPAD, D_S), jnp.float32),
    ]

    @functools.partial(
        pl.kernel,
        out_type=out_type,
        mesh=_vmesh,
        compiler_params=_sc_params,
        scratch_types=[
            pltpu.VMEM((NW, NPW), jnp.int32),
            pltpu.VMEM((NPW,), jnp.int32),
            pltpu.VMEM((NPW,), jnp.float32),
            pltpu.VMEM((NPW,), jnp.int32),
            pltpu.VMEM((NPW,), jnp.float32),
            pltpu.VMEM((NPW, D_E), jnp.float32),
            pltpu.VMEM((NPW, D_S), jnp.float32),
        ],
    )
    def k(pr_hbm, pg_hbm, src_hbm, dst_hbm, t_hbm, e_hbm, si_hbm, sj_hbm,
          mr_hbm, tr_hbm, er_hbm, ur_hbm, mg_hbm, tg_hbm, eg_hbm, ug_hbm,
          part_v, k_v, mask_v, oid_v, tsel_v, esel_v, usel_v):
        wid = lax.axis_index("s") * 2 + lax.axis_index("c")
        base = wid * NPW

        def side(p_hbm, peer_hbm, table_hbm, m_hbm, ts_hbm, es_hbm, us_hbm):
            pltpu.sync_copy(p_hbm.at[:, pl.ds(base, NPW)], part_v)

            @pl.loop(0, NPW, step=16)
            def _(r):
                m = part_v[0, pl.ds(r, 16)]
                for w in range(1, NW):
                    m = jnp.maximum(m, part_v[w, pl.ds(r, 16)])
                mask_v[pl.ds(r, 16)] = jnp.where(
                    m >= 0, jnp.float32(1.0), jnp.float32(0.0))
                k_v[pl.ds(r, 16)] = jnp.maximum(m, 0)

            pltpu.sync_copy(peer_hbm.at[k_v], oid_v)
            pltpu.sync_copy(t_hbm.at[k_v], tsel_v)
            pltpu.sync_copy(e_hbm.at[k_v], esel_v)
            pltpu.sync_copy(table_hbm.at[oid_v], usel_v)

            pltpu.sync_copy(mask_v, m_hbm.at[pl.ds(base, NPW)])
            pltpu.sync_copy(tsel_v, ts_hbm.at[pl.ds(base, NPW)])
            pltpu.sync_copy(esel_v, es_hbm.at[pl.ds(base, NPW)])
            pltpu.sync_copy(usel_v, us_hbm.at[pl.ds(base, NPW)])

        side(pr_hbm, dst_hbm, sj_hbm, mr_hbm, tr_hbm, er_hbm, ur_hbm)
        side(pg_hbm, src_hbm, si_hbm, mg_hbm, tg_hbm, eg_hbm, ug_hbm)

    return k(part_r, part_g, edge_src, edge_dst, t, e, si, sj)


ROWS_PER_BLOCK = 1000
_GRID = N_NODES // ROWS_PER_BLOCK


def _gru_body(si_ref, sj_ref, ur_ref, ug_ref, tr_ref, tg_ref, er_ref, eg_ref,
              mr_ref, mg_ref, bf_ref, ph_ref, wih_ref, whh_ref, bih_ref,
              bhh_ref, osi_ref, osj_ref):
    wih = wih_ref[...]
    whh = whh_ref[...]
    bih = bih_ref[...]
    bhh = bhh_ref[...]
    bf = bf_ref[...]
    ph = ph_ref[...]

    def side(h, u, tsel, esel, m):
        time_emb = jnp.cos(tsel * bf + ph) * m
        x = jnp.concatenate([u * m, h * m, time_emb, esel * m], axis=1)
        gi = lax.dot_general(x, wih, (((1,), (1,)), ((), ())),
                             preferred_element_type=jnp.float32) + bih
        gh = lax.dot_general(h, whh, (((1,), (1,)), ((), ())),
                             preferred_element_type=jnp.float32) + bhh
        r = jax.nn.sigmoid(gi[:, :D_S] + gh[:, :D_S])
        z = jax.nn.sigmoid(gi[:, D_S:2 * D_S] + gh[:, D_S:2 * D_S])
        n = jnp.tanh(gi[:, 2 * D_S:] + r * gh[:, 2 * D_S:])
        return (1.0 - z) * n + z * h

    osi_ref[...] = side(si_ref[...], ur_ref[...], tr_ref[...], er_ref[...],
                        mr_ref[...])
    osj_ref[...] = side(sj_ref[...], ug_ref[...], tg_ref[...], eg_ref[...],
                        mg_ref[...])


def _row_spec(cols):
    return pl.BlockSpec((ROWS_PER_BLOCK, cols), lambda i: (i, 0))


def _full_spec(rows, cols):
    return pl.BlockSpec((rows, cols), lambda i: (0, 0))


def _tc_gru(si, sj, ur, ug, tr2, tg2, er, eg, mr2, mg2, bf2, ph2,
            w_ih, w_hh, bih2, bhh2):
    return pl.pallas_call(
        _gru_body,
        grid=(_GRID,),
        in_specs=[
            _row_spec(D_S), _row_spec(D_S),
            _row_spec(D_S), _row_spec(D_S),
            _row_spec(1), _row_spec(1),
            _row_spec(D_E), _row_spec(D_E),
            _row_spec(1), _row_spec(1),
            _full_spec(1, D_T), _full_spec(1, D_T),
            _full_spec(3 * D_S, 2 * D_S + D_T + D_E),
            _full_spec(3 * D_S, D_S),
            _full_spec(1, 3 * D_S), _full_spec(1, 3 * D_S),
        ],
        out_specs=[_row_spec(D_S), _row_spec(D_S)],
        out_shape=[
            jax.ShapeDtypeStruct((N_NODES, D_S), jnp.float32),
            jax.ShapeDtypeStruct((N_NODES, D_S), jnp.float32),
        ],
    )(si, sj, ur, ug, tr2, tg2, er, eg, mr2, mg2, bf2, ph2,
      w_ih, w_hh, bih2, bhh2)


def kernel(si, sj, t, e, edge_src, edge_dst, basis_freq, phase,
           w_ih, w_hh, b_ih, b_hh):
    part_r, part_g = _sc_scatter(edge_src, edge_dst)
    mr, tr, er, ur, mg, tg, eg, ug = _sc_gather(
        part_r, part_g, edge_src, edge_dst, t, e, si, sj)

    n = N_NODES
    tr2 = tr[:n].reshape(n, 1)
    tg2 = tg[:n].reshape(n, 1)
    mr2 = mr[:n].reshape(n, 1)
    mg2 = mg[:n].reshape(n, 1)
    si_new, sj_new = _tc_gru(
        si, sj, ur[:n], ug[:n], tr2, tg2, er[:n], eg[:n], mr2, mg2,
        basis_freq.reshape(1, D_T), phase.reshape(1, D_T),
        w_ih, w_hh, b_ih.reshape(1, 3 * D_S), b_hh.reshape(1, 3 * D_S))
    return (si_new, sj_new)


# speedup vs baseline: 14.8398x; 1.8138x over previous
import dataclasses
import functools

import jax
import jax.numpy as jnp
from jax import lax
from jax.experimental import pallas as pl
from jax.experimental.pallas import tpu as pltpu
from jax.experimental.pallas import tpu_sc as plsc

N_NODES = 5000
E_TOTAL = 160000
D_S = 128
D_T = 16
D_E = 16

NPAD = 5120
NSUB = 16
EPT = E_TOTAL // NSUB
NPT = NPAD // NSUB

_vmesh = plsc.VectorSubcoreMesh(core_axis_name="c", subcore_axis_name="s")

_sc_params = pltpu.CompilerParams()
if "needs_layout_passes" in pltpu.CompilerParams.__dataclass_fields__:
    _sc_params = dataclasses.replace(_sc_params, needs_layout_passes=False)
if "use_tc_tiling_on_sc" in pltpu.CompilerParams.__dataclass_fields__:
    _sc_params = dataclasses.replace(_sc_params, use_tc_tiling_on_sc=False)


@jax.jit
def _sc_fused(edges_cat, t, ev, table_cat):
    out_type = [
        jax.ShapeDtypeStruct((2 * NPAD, D_S), jnp.float32),
        jax.ShapeDtypeStruct((2 * NPAD, D_S), jnp.float32),
        jax.ShapeDtypeStruct((2, NPAD), jnp.float32),
    ]

    @functools.partial(
        pl.kernel,
        out_type=out_type,
        mesh=_vmesh,
        compiler_params=_sc_params,
        scratch_types=[
            pltpu.VMEM((EPT,), jnp.int32),
            pltpu.VMEM((NPAD,), jnp.int32),
            pltpu.VMEM((2, 32), jnp.int32),
            pltpu.VMEM_SHARED((NSUB, NPAD), jnp.int32),
            pltpu.VMEM((NSUB, NPT), jnp.int32),
            pltpu.VMEM((NPT,), jnp.int32),
            pltpu.VMEM((NPT,), jnp.int32),
            pltpu.VMEM((NPT,), jnp.float32),
            pltpu.VMEM((NPT,), jnp.int32),
            pltpu.VMEM((NPT,), jnp.float32),
            pltpu.VMEM((D_E * NPT,), jnp.int32),
            pltpu.VMEM((D_E * NPT,), jnp.float32),
            pltpu.VMEM((NPT, 2 * D_E), jnp.float32),
            pltpu.VMEM((NPT, D_S), jnp.float32),
            pltpu.SemaphoreType.DMA,
            pltpu.SemaphoreType.DMA,
        ],
    )
    def k(edges_hbm, t_hbm, ev_hbm, table_hbm, co_hbm, u_hbm, tT_hbm,
          ec_v, tbl_v, shift_v, part_sh, mrg_v, k_v, pidx_v, mask_v, oid_v,
          tsel_v, eidx_v, et_v, esel_v, usel_v, sem, semp):
        c = lax.axis_index("c")
        s = lax.axis_index("s")
        iota16 = lax.iota(jnp.int32, 16)
        neg = jnp.full((16,), -1, jnp.int32)

        base_e = c * E_TOTAL + s * EPT
        ecopy = pltpu.async_copy(edges_hbm.at[pl.ds(base_e, EPT)], ec_v, sem)

        @pl.loop(0, NPAD, step=16)
        def _(i):
            tbl_v[pl.ds(i, 16)] = neg

        for q in range(2):
            shift_v[q, pl.ds(16, 16)] = neg
        ecopy.wait()

        def scatter_one(q, k_):
            idxv = ec_v[pl.ds(k_, 16)]
            val = s * EPT + k_ + iota16
            comp = (idxv << 18) | val
            sk, sval = plsc.sort_key_val(comp, val)
            shift_v[q, pl.ds(0, 16)] = sk
            nxt = shift_v[q, pl.ds(1, 16)]
            is_last = (sk >> 18) != (nxt >> 18)
            plsc.store_scatter(tbl_v, [sk >> 18], sval, mask=is_last)

        @pl.loop(0, EPT - 16, step=32)
        def _(k_):
            scatter_one(0, k_)
            scatter_one(1, k_ + 16)

        scatter_one(0, EPT - 16)

        pltpu.sync_copy(tbl_v, part_sh.at[s])
        plsc.subcore_barrier()

        npt_base = s * NPT
        pltpu.sync_copy(part_sh.at[:, pl.ds(npt_base, NPT)], mrg_v)

        peer_off = (1 - c) * E_TOTAL

        @pl.loop(0, NPT, step=16)
        def _(r):
            m = mrg_v[0, pl.ds(r, 16)]
            for w in range(1, NSUB):
                m = jnp.maximum(m, mrg_v[w, pl.ds(r, 16)])
            mask_v[pl.ds(r, 16)] = jnp.where(
                m >= 0, jnp.float32(1.0), jnp.float32(0.0))
            kk = jnp.maximum(m, 0)
            k_v[pl.ds(r, 16)] = kk
            pidx_v[pl.ds(r, 16)] = kk + peer_off
            for cc in range(D_E):
                eidx_v[pl.ds(cc * NPT + r, 16)] = kk + cc * E_TOTAL

        pc = pltpu.async_copy(edges_hbm.at[pidx_v], oid_v, semp)
        tc_ = pltpu.async_copy(t_hbm.at[k_v], tsel_v, sem)
        ec2 = pltpu.async_copy(ev_hbm.at[eidx_v], et_v, sem)
        pc.wait()

        @pl.loop(0, NPT, step=16)
        def _(r):
            oid_v[pl.ds(r, 16)] = oid_v[pl.ds(r, 16)] + c * N_NODES

        uc = pltpu.async_copy(table_hbm.at[oid_v], usel_v, sem)
        tc_.wait()
        ec2.wait()

        @pl.loop(0, NPT)
        def _(j):
            esel_v[j, pl.ds(0, D_E)] = plsc.load_gather(
                et_v, [iota16 * NPT + j])

        @pl.loop(0, NPT, step=16)
        def _(r):
            plsc.store_scatter(
                esel_v, [r + iota16, jnp.full((16,), D_E, jnp.int32)],
                mask_v[pl.ds(r, 16)])

        out_base = c * NPAD + npt_base
        oc1 = pltpu.async_copy(
            esel_v, co_hbm.at[pl.ds(out_base, NPT), pl.ds(0, 2 * D_E)], sem)
        oc2 = pltpu.async_copy(tsel_v, tT_hbm.at[c, pl.ds(npt_base, NPT)],
                               sem)
        uc.wait()
        oc3 = pltpu.async_copy(usel_v, u_hbm.at[pl.ds(out_base, NPT)], sem)
        oc1.wait()
        oc2.wait()
        oc3.wait()

    return k(edges_cat, t, ev, table_cat)


ROWS_PER_BLOCK = 1024
_GRID = (N_NODES + ROWS_PER_BLOCK - 1) // ROWS_PER_BLOCK
_GOFF = NPAD // ROWS_PER_BLOCK


def _gru_body(si_ref, sj_ref, ur_ref, ug_ref, cr_ref, cg_ref, tT_ref,
              bf_ref, ph_ref, wa_ref, w3_ref, whh_ref, bih_ref,
              bhh_ref, osi_ref, osj_ref):
    wa = wa_ref[...].astype(jnp.bfloat16)
    w3 = w3_ref[...].astype(jnp.bfloat16)
    whh = whh_ref[...].astype(jnp.bfloat16)
    bih = bih_ref[...]
    bhh = bhh_ref[...]
    bf = bf_ref[...]
    ph = ph_ref[...]

    def side(h, u, combo, tT):
        esel = combo[:, :D_E]
        m = combo[:, D_E:D_E + 1]
        teT = jnp.cos(bf * tT + ph)
        x3 = jnp.concatenate([u, h, esel], axis=1)
        gi = (lax.dot_general(x3.astype(jnp.bfloat16), wa,
                              (((1,), (1,)), ((), ())),
                              preferred_element_type=jnp.float32)
              + lax.dot_general(teT.astype(jnp.bfloat16), w3,
                                (((0,), (1,)), ((), ())),
                                preferred_element_type=jnp.float32)) * m + bih
        gh = lax.dot_general(h.astype(jnp.bfloat16), whh,
                             (((1,), (1,)), ((), ())),
                             preferred_element_type=jnp.float32) + bhh
        r = jax.nn.sigmoid(gi[:, :D_S] + gh[:, :D_S])
        z = jax.nn.sigmoid(gi[:, D_S:2 * D_S] + gh[:, D_S:2 * D_S])
        n = jnp.tanh(gi[:, 2 * D_S:] + r * gh[:, 2 * D_S:])
        return (1.0 - z) * n + z * h

    osi_ref[...] = side(si_ref[...], ur_ref[...], cr_ref[...],
                        tT_ref[0:1, :])
    osj_ref[...] = side(sj_ref[...], ug_ref[...], cg_ref[...],
                        tT_ref[1:2, :])


def _row_spec(cols):
    return pl.BlockSpec((ROWS_PER_BLOCK, cols), lambda i: (i, 0))


def _row_spec_hi(cols):
    return pl.BlockSpec((ROWS_PER_BLOCK, cols), lambda i: (i + _GOFF, 0))


def _full_spec(rows, cols):
    return pl.BlockSpec((rows, cols), lambda i: (0, 0))


def _tc_gru(si, sj, u2, c2, tT, bf2, ph2, wa, w3, w_hh, bih2, bhh2):
    return pl.pallas_call(
        _gru_body,
        grid=(_GRID,),
        in_specs=[
            _row_spec(D_S), _row_spec(D_S),
            _row_spec(D_S), _row_spec_hi(D_S),
            _row_spec(D_S), _row_spec_hi(D_S),
            pl.BlockSpec((2, ROWS_PER_BLOCK), lambda i: (0, i)),
            _full_spec(D_T, 1), _full_spec(D_T, 1),
            _full_spec(3 * D_S, 2 * D_S + D_E),
            _full_spec(3 * D_S, D_T),
            _full_spec(3 * D_S, D_S),
            _full_spec(1, 3 * D_S), _full_spec(1, 3 * D_S),
        ],
        out_specs=[_row_spec(D_S), _row_spec(D_S)],
        out_shape=[
            jax.ShapeDtypeStruct((N_NODES, D_S), jnp.float32),
            jax.ShapeDtypeStruct((N_NODES, D_S), jnp.float32),
        ],
    )(si, sj, u2, u2, c2, c2, tT, bf2, ph2, wa, w3, w_hh, bih2, bhh2)


def kernel(si, sj, t, e, edge_src, edge_dst, basis_freq, phase,
           w_ih, w_hh, b_ih, b_hh):
    edges_cat = jnp.concatenate([edge_src, edge_dst])
    table_cat = jnp.concatenate([sj, si], axis=0)
    ev = e.T.reshape(-1)
    c2, u2, tT = _sc_fused(edges_cat, t, ev, table_cat)

    wa = jnp.concatenate([w_ih[:, :2 * D_S], w_ih[:, 2 * D_S + D_T:]], axis=1)
    w3 = w_ih[:, 2 * D_S:2 * D_S + D_T]
    si_new, sj_new = _tc_gru(
        si, sj, u2, c2, tT,
        basis_freq.reshape(D_T, 1), phase.reshape(D_T, 1),
        wa, w3, w_hh, b_ih.reshape(1, 3 * D_S), b_hh.reshape(1, 3 * D_S))
    return (si_new, sj_new)


# speedup vs baseline: 16.3503x; 1.1018x over previous
import dataclasses
import functools

import jax
import jax.numpy as jnp
from jax import lax
from jax.experimental import pallas as pl
from jax.experimental.pallas import tpu as pltpu
from jax.experimental.pallas import tpu_sc as plsc

N_NODES = 5000
E_TOTAL = 160000
D_S = 128
D_T = 16
D_E = 16

NW = 32
EPW = E_TOTAL // NW
NPAD = 5120
NPW = NPAD // NW

_vmesh = plsc.VectorSubcoreMesh(core_axis_name="c", subcore_axis_name="s")

_sc_params = pltpu.CompilerParams()
if "needs_layout_passes" in pltpu.CompilerParams.__dataclass_fields__:
    _sc_params = dataclasses.replace(_sc_params, needs_layout_passes=False)
if "use_tc_tiling_on_sc" in pltpu.CompilerParams.__dataclass_fields__:
    _sc_params = dataclasses.replace(_sc_params, use_tc_tiling_on_sc=False)


@jax.jit
def _sc_scatter(edge_src, edge_dst):
    @functools.partial(
        pl.kernel,
        out_type=[
            jax.ShapeDtypeStruct((NW * NPAD,), jnp.int32),
            jax.ShapeDtypeStruct((NW * NPAD,), jnp.int32),
        ],
        mesh=_vmesh,
        compiler_params=_sc_params,
        scratch_types=[
            pltpu.VMEM((EPW + 16,), jnp.int32),
            pltpu.VMEM((EPW + 16,), jnp.int32),
            pltpu.VMEM((NPAD,), jnp.int32),
            pltpu.VMEM((NPAD,), jnp.int32),
            pltpu.VMEM((4, 32), jnp.int32),
        ],
    )
    def k(src_hbm, dst_hbm, pr_hbm, pg_hbm, src_v, dst_v, lr_v, lg_v,
          shift_v):
        wid = lax.axis_index("s") * 2 + lax.axis_index("c")
        base = wid * EPW
        pltpu.sync_copy(src_hbm.at[pl.ds(base, EPW)],
                        src_v.at[pl.ds(0, EPW)])
        pltpu.sync_copy(dst_hbm.at[pl.ds(base, EPW)],
                        dst_v.at[pl.ds(0, EPW)])

        neg = jnp.full((16,), -1, jnp.int32)

        @pl.loop(0, NPAD, step=16)
        def _(i):
            lr_v[pl.ds(i, 16)] = neg
            lg_v[pl.ds(i, 16)] = neg

        iota16 = lax.iota(jnp.int32, 16)
        for q in range(4):
            shift_v[q, pl.ds(16, 16)] = neg

        def scatter_one(arr_ref, q, idxv, val):
            comp = (idxv << 18) | val
            sk, sval = plsc.sort_key_val(comp, val)
            shift_v[q, pl.ds(0, 16)] = sk
            nxt = shift_v[q, pl.ds(1, 16)]
            is_last = (sk >> 18) != (nxt >> 18)
            plsc.store_scatter(arr_ref, [sk >> 18], sval, mask=is_last)

        def do_vreg(k_, qbase, valid):
            sv = src_v[pl.ds(k_, 16)]
            dv = dst_v[pl.ds(k_, 16)]
            val = base + k_ + iota16
            if valid is not None:
                sv = jnp.where(valid, sv, NPAD - 1)
                dv = jnp.where(valid, dv, NPAD - 1)
                val = jnp.where(valid, val, 0)
            scatter_one(lr_v, qbase, sv, val)
            scatter_one(lg_v, qbase + 1, dv, val)

        tail_n = EPW % 32
        main_hi = EPW - tail_n

        @pl.loop(0, main_hi, step=32)
        def _(k_):
            do_vreg(k_, 0, None)
            do_vreg(k_ + 16, 2, None)

        if tail_n >= 16:
            do_vreg(main_hi, 0, None)
        rem = EPW % 16
        if rem:
            do_vreg(EPW - rem, 2, iota16 < rem)

        pltpu.sync_copy(lr_v, pr_hbm.at[pl.ds(wid * NPAD, NPAD)])
        pltpu.sync_copy(lg_v, pg_hbm.at[pl.ds(wid * NPAD, NPAD)])

    return k(edge_src, edge_dst)


@jax.jit
def _sc_gather(part_r, part_g, edge_src, edge_dst, t, ev, si, sj):
    out_type = [
        jax.ShapeDtypeStruct((NPAD, D_S), jnp.float32),
        jax.ShapeDtypeStruct((NPAD, D_S), jnp.float32),
        jax.ShapeDtypeStruct((NPAD, D_S), jnp.float32),
        jax.ShapeDtypeStruct((NPAD, D_S), jnp.float32),
        jax.ShapeDtypeStruct((2, NPAD), jnp.float32),
    ]

    @functools.partial(
        pl.kernel,
        out_type=out_type,
        mesh=_vmesh,
        compiler_params=_sc_params,
        scratch_types=[
            pltpu.VMEM((2, NW, NPW), jnp.int32),
            pltpu.VMEM((2, NPW), jnp.int32),
            pltpu.VMEM((2, NPW), jnp.float32),
            pltpu.VMEM((2, NPW), jnp.int32),
            pltpu.VMEM((2, NPW), jnp.float32),
            pltpu.VMEM((2, D_E * NPW), jnp.int32),
            pltpu.VMEM((2, D_E * NPW), jnp.float32),
            pltpu.VMEM((2, NPW, 2 * D_E), jnp.float32),
            pltpu.VMEM((2, NPW, D_S), jnp.float32),
            pltpu.SemaphoreType.DMA,
            pltpu.SemaphoreType.DMA,
            pltpu.SemaphoreType.DMA,
        ],
    )
    def k(pr_hbm, pg_hbm, src_hbm, dst_hbm, t_hbm, ev_hbm, si_hbm, sj_hbm,
          er_hbm, ur_hbm, eg_hbm, ug_hbm, tT_hbm,
          part_v, k_v, mask_v, oid_v, tsel_v, eidx_v, et_v, esel_v, usel_v,
          sem, semp0, semp1):
        wid = lax.axis_index("s") * 2 + lax.axis_index("c")
        base = wid * NPW
        iota16 = lax.iota(jnp.int32, 16)
        sides = [
            (0, pr_hbm, dst_hbm, sj_hbm, er_hbm, ur_hbm, semp0),
            (1, pg_hbm, src_hbm, si_hbm, eg_hbm, ug_hbm, semp1),
        ]

        part_copies = [
            pltpu.async_copy(p_hbm.at[pl.ds(w * NPAD + base, NPW)],
                             part_v.at[i, w], sem)
            for (i, p_hbm, *_rest) in sides
            for w in range(NW)
        ]
        for c_ in part_copies:
            c_.wait()

        peer_copies = []
        misc_copies = []
        for (i, _p, peer_hbm, _tab, _es, _us, semp) in sides:
            @pl.loop(0, NPW, step=16)
            def _(r, i=i):
                m = part_v[i, 0, pl.ds(r, 16)]
                for w in range(1, NW):
                    m = jnp.maximum(m, part_v[i, w, pl.ds(r, 16)])
                mask_v[i, pl.ds(r, 16)] = jnp.where(
                    m >= 0, jnp.float32(1.0), jnp.float32(0.0))
                kk = jnp.maximum(m, 0)
                k_v[i, pl.ds(r, 16)] = kk
                for c in range(D_E):
                    eidx_v[i, pl.ds(c * NPW + r, 16)] = kk + c * E_TOTAL

            peer_copies.append(
                pltpu.async_copy(peer_hbm.at[k_v.at[i]], oid_v.at[i], semp))
            misc_copies.append(
                pltpu.async_copy(t_hbm.at[k_v.at[i]], tsel_v.at[i], sem))
            misc_copies.append(
                pltpu.async_copy(ev_hbm.at[eidx_v.at[i]], et_v.at[i], sem))

        table_copies = []
        for idx, (i, _p, _peer, table_hbm, _es, _us, _semp) in enumerate(sides):
            peer_copies[idx].wait()
            table_copies.append(
                pltpu.async_copy(table_hbm.at[oid_v.at[i]], usel_v.at[i],
                                 _semp))
        for c_ in misc_copies:
            c_.wait()

        out_copies = []
        for (i, _p, _peer, _tab, es_hbm, us_hbm, _semp) in sides:
            @pl.loop(0, NPW)
            def _(j, i=i):
                esel_v[i, j, pl.ds(0, D_E)] = plsc.load_gather(
                    et_v.at[i], [iota16 * NPW + j])

            @pl.loop(0, NPW, step=16)
            def _(r, i=i):
                rows = r + iota16
                plsc.store_scatter(
                    esel_v.at[i], [rows, jnp.full((16,), D_E, jnp.int32)],
                    mask_v[i, pl.ds(r, 16)])

            out_copies.append(
                pltpu.async_copy(
                    esel_v.at[i],
                    es_hbm.at[pl.ds(base, NPW), pl.ds(0, 2 * D_E)], sem))
            out_copies.append(
                pltpu.async_copy(tsel_v.at[i],
                                 tT_hbm.at[i, pl.ds(base, NPW)], sem))

        for c_ in table_copies:
            c_.wait()
        for (i, _p, _peer, _tab, _es, us_hbm, _semp) in sides:
            out_copies.append(
                pltpu.async_copy(usel_v.at[i], us_hbm.at[pl.ds(base, NPW)],
                                 sem))
        for c_ in out_copies:
            c_.wait()

    return k(part_r, part_g, edge_src, edge_dst, t, ev, si, sj)


ROWS_PER_BLOCK = 1024
_GRID = (N_NODES + ROWS_PER_BLOCK - 1) // ROWS_PER_BLOCK


def _gru_body(si_ref, sj_ref, ur_ref, ug_ref, cr_ref, cg_ref, tT_ref,
              bf_ref, ph_ref, wa_ref, w3_ref, whh_ref, bih_ref,
              bhh_ref, osi_ref, osj_ref):
    wa = wa_ref[...].astype(jnp.bfloat16)
    w3 = w3_ref[...].astype(jnp.bfloat16)
    whh = whh_ref[...].astype(jnp.bfloat16)
    bih = bih_ref[...]
    bhh = bhh_ref[...]
    bf = bf_ref[...]
    ph = ph_ref[...]

    def side(h, u, combo, tT):
        esel = combo[:, :D_E]
        m = combo[:, D_E:D_E + 1]
        teT = jnp.cos(bf * tT + ph)
        x3 = jnp.concatenate([u, h, esel], axis=1)
        gi = (lax.dot_general(x3.astype(jnp.bfloat16), wa,
                              (((1,), (1,)), ((), ())),
                              preferred_element_type=jnp.float32)
              + lax.dot_general(teT.astype(jnp.bfloat16), w3,
                                (((0,), (1,)), ((), ())),
                                preferred_element_type=jnp.float32)) * m + bih
        gh = lax.dot_general(h.astype(jnp.bfloat16), whh,
                             (((1,), (1,)), ((), ())),
                             preferred_element_type=jnp.float32) + bhh
        r = jax.nn.sigmoid(gi[:, :D_S] + gh[:, :D_S])
        z = jax.nn.sigmoid(gi[:, D_S:2 * D_S] + gh[:, D_S:2 * D_S])
        n = jnp.tanh(gi[:, 2 * D_S:] + r * gh[:, 2 * D_S:])
        return (1.0 - z) * n + z * h

    osi_ref[...] = side(si_ref[...], ur_ref[...], cr_ref[...],
                        tT_ref[0:1, :])
    osj_ref[...] = side(sj_ref[...], ug_ref[...], cg_ref[...],
                        tT_ref[1:2, :])


def _row_spec(cols):
    return pl.BlockSpec((ROWS_PER_BLOCK, cols), lambda i: (i, 0))


def _full_spec(rows, cols):
    return pl.BlockSpec((rows, cols), lambda i: (0, 0))


def _tc_gru(si, sj, ur, ug, cr, cg, tT, bf2, ph2, wa, w3, w_hh, bih2, bhh2):
    return pl.pallas_call(
        _gru_body,
        grid=(_GRID,),
        in_specs=[
            _row_spec(D_S), _row_spec(D_S),
            _row_spec(D_S), _row_spec(D_S),
            _row_spec(D_S), _row_spec(D_S),
            pl.BlockSpec((2, ROWS_PER_BLOCK), lambda i: (0, i)),
            _full_spec(D_T, 1), _full_spec(D_T, 1),
            _full_spec(3 * D_S, 2 * D_S + D_E),
            _full_spec(3 * D_S, D_T),
            _full_spec(3 * D_S, D_S),
            _full_spec(1, 3 * D_S), _full_spec(1, 3 * D_S),
        ],
        out_specs=[_row_spec(D_S), _row_spec(D_S)],
        out_shape=[
            jax.ShapeDtypeStruct((N_NODES, D_S), jnp.float32),
            jax.ShapeDtypeStruct((N_NODES, D_S), jnp.float32),
        ],
    )(si, sj, ur, ug, cr, cg, tT, bf2, ph2, wa, w3, w_hh, bih2, bhh2)


def kernel(si, sj, t, e, edge_src, edge_dst, basis_freq, phase,
           w_ih, w_hh, b_ih, b_hh):
    part_r, part_g = _sc_scatter(edge_src, edge_dst)
    ev = e.T.reshape(-1)
    cr, ur, cg, ug, tT = _sc_gather(
        part_r, part_g, edge_src, edge_dst, t, ev, si, sj)

    wa = jnp.concatenate([w_ih[:, :2 * D_S], w_ih[:, 2 * D_S + D_T:]], axis=1)
    w3 = w_ih[:, 2 * D_S:2 * D_S + D_T]
    si_new, sj_new = _tc_gru(
        si, sj, ur, ug, cr, cg, tT,
        basis_freq.reshape(D_T, 1), phase.reshape(D_T, 1),
        wa, w3, w_hh, b_ih.reshape(1, 3 * D_S), b_hh.reshape(1, 3 * D_S))
    return (si_new, sj_new)
